# baseline (speedup 1.0000x reference)
import jax
import jax.numpy as jnp
from jax import lax
from jax.experimental import pallas as pl
from jax.experimental.pallas import tpu as pltpu
from jax.experimental.pallas import tpu_sc as plsc

_B, _N, _F, _OUT, _E = 8, 10000, 128, 64, 320000
_G = 4
_GW = 2 * _OUT
_NS = 16
_CH = 80
_NCH = 250
_EP = _NS * _NCH * _CH
_RPT = 632
_NB = 2000


def _proj_body(x_ref, w_ref, p0_ref, p1_ref, p2_ref):
    w = w_ref[...]
    y0 = jnp.dot(x_ref[0], w, preferred_element_type=jnp.float32)
    y1 = jnp.dot(x_ref[1], w, preferred_element_type=jnp.float32)
    for k, ref in enumerate((p0_ref, p1_ref, p2_ref)):
        ref[0, :, 0, :] = y0[:, k * _OUT:(k + 1) * _OUT]
        ref[0, :, 1, :] = y1[:, k * _OUT:(k + 1) * _OUT]


def _project(x, wc):
    pshape = jax.ShapeDtypeStruct((_G, _N, 2, _OUT), jnp.float32)
    pspec = pl.BlockSpec((1, _NB, 2, _OUT), lambda g, n: (g, n, 0, 0))
    return pl.pallas_call(
        _proj_body,
        grid=(_G, _N // _NB),
        in_specs=[pl.BlockSpec((2, _NB, _F), lambda g, n: (g, n, 0)),
                  pl.BlockSpec((_F, 3 * _OUT), lambda g, n: (0, 0))],
        out_specs=[pspec, pspec, pspec],
        out_shape=[pshape, pshape, pshape],
    )(x, wc)


def _lane_splat(vec, l):
    return lax.gather(
        vec, jnp.full((16, 1), l, jnp.int32),
        lax.GatherDimensionNumbers(offset_dims=(), collapsed_slice_dims=(0,),
                                   start_index_map=(0,)),
        (1,), mode=lax.GatherScatterMode.PROMISE_IN_BOUNDS)


def _spmm_body(v_hbm, init_hbm, src_hbm, dst_hbm, w_hbm, out_hbm,
               src_c, dst_c, w_c, rows_v, acc,
               isem, gsem, ssem):
    c = lax.axis_index("c")
    s = lax.axis_index("s")
    row0 = jnp.minimum(s * _RPT, _N - _RPT)

    def _start_idx(g, j, q):
        pltpu.async_copy(src_hbm.at[g, s, j], src_c.at[q], isem[q])
        pltpu.async_copy(dst_hbm.at[s, j], dst_c.at[q], isem[q])
        pltpu.async_copy(w_hbm.at[s, j], w_c.at[q], isem[q])

    def _wait_idx(g, j, q):
        pltpu.make_async_copy(src_hbm.at[g, s, j], src_c.at[q], isem[q]).wait()
        pltpu.make_async_copy(dst_hbm.at[s, j], dst_c.at[q], isem[q]).wait()
        pltpu.make_async_copy(w_hbm.at[s, j], w_c.at[q], isem[q]).wait()

    for gi in range(2):
        g = c * 2 + gi
        pltpu.sync_copy(init_hbm.at[pl.ds(g * _N + row0, _RPT)],
                        acc.at[pl.ds(row0, _RPT)])
        plsc.subcore_barrier()

        for jj in range(4):
            _start_idx(g, jj, jj)
        for jj in range(2):
            _wait_idx(g, jj, jj)
            pltpu.async_copy(v_hbm.at[src_c.at[jj]], rows_v.at[jj], gsem[jj])

        def _pos(i, carry):
            for qq in range(8):
                m = 8 * i + qq
                r = qq % 4

                @pl.when((m >= 2) & (m <= _NCH + 1))
                def _():
                    pltpu.make_async_copy(
                        rows_v.at[(qq + 2) % 4],
                        acc.at[dst_c.at[(qq + 6) % 8]],
                        ssem[(qq + 2) % 4]).wait()

                @pl.when(m <= _NCH - 5)
                def _():
                    _start_idx(g, m + 4, (qq + 4) % 8)

                @pl.when(m <= _NCH - 3)
                def _():
                    _wait_idx(g, m + 2, (qq + 2) % 8)
                    pltpu.async_copy(v_hbm.at[src_c.at[(qq + 2) % 8]],
                                     rows_v.at[(qq + 2) % 4],
                                     gsem[(qq + 2) % 4])

                @pl.when(m <= _NCH - 1)
                def _():
                    pltpu.make_async_copy(v_hbm.at[src_c.at[qq % 8]],
                                          rows_v.at[r], gsem[r]).wait()

                    wrow = w_c.at[qq % 8]

                    def _edge16(e16, carry2):
                        base = e16 * 16
                        wv16 = wrow[pl.ds(base, 16)]
                        for l in range(16):
                            wsp = _lane_splat(wv16, l)
                            for t in range(_GW // 16):
                                sl = pl.ds(t * 16, 16)
                                rows_v[r, base + l, sl] = (
                                    rows_v[r, base + l, sl] * wsp)
                        return carry2

                    lax.fori_loop(0, _CH // 16, _edge16, 0)
                    pltpu.async_copy(rows_v.at[r], acc.at[dst_c.at[qq % 8]],
                                     ssem[r], add=True)
            return carry

        lax.fori_loop(0, (_NCH + 2 + 7) // 8, _pos, 0)
        plsc.subcore_barrier()
        pltpu.sync_copy(acc.at[pl.ds(row0, _RPT)],
                        out_hbm.at[pl.ds(g * _N + row0, _RPT)])
        plsc.subcore_barrier()


_spmm = pl.kernel(
    _spmm_body,
    out_type=jax.ShapeDtypeStruct((_G * _N, _GW), jnp.float32),
    mesh=plsc.VectorSubcoreMesh(core_axis_name="c", subcore_axis_name="s",
                                num_cores=2, num_subcores=_NS),
    scratch_types=[
        pltpu.VMEM((8, _CH), jnp.int32),
        pltpu.VMEM((8, _CH), jnp.int32),
        pltpu.VMEM((8, _CH), jnp.float32),
        pltpu.VMEM((4, _CH, _GW), jnp.float32),
        pltpu.VMEM_SHARED((_N, _GW), jnp.float32),
        [pltpu.SemaphoreType.DMA] * 8,
        [pltpu.SemaphoreType.DMA] * 4,
        [pltpu.SemaphoreType.DMA] * 4,
    ],
    compiler_params=pltpu.CompilerParams(needs_layout_passes=False),
)


def _fin_body(u_ref, b_ref, o_ref):
    u = u_ref[0]
    bias = b_ref[0]
    z0 = u[:, 0, :] + bias
    z1 = u[:, 1, :] + bias
    o_ref[0] = jnp.where(z0 > 0, z0, jnp.exp(jnp.minimum(z0, 0.0)) - 1.0)
    o_ref[1] = jnp.where(z1 > 0, z1, jnp.exp(jnp.minimum(z1, 0.0)) - 1.0)


def _finish(u, bias):
    return pl.pallas_call(
        _fin_body,
        grid=(_G, _N // _NB),
        in_specs=[pl.BlockSpec((1, _NB, 2, _OUT), lambda g, n: (g, n, 0, 0)),
                  pl.BlockSpec((1, _OUT), lambda g, n: (0, 0))],
        out_specs=pl.BlockSpec((2, _NB, _OUT), lambda g, n: (g, n, 0)),
        out_shape=jax.ShapeDtypeStruct((_B, _N, _OUT), jnp.float32),
    )(u, bias)


def kernel(inputs, edge_index, edge_weight, W, b):
    x = inputs.reshape(_B, _N, _F)
    w0, w1, w2 = W[0::3], W[1::3], W[2::3]
    wc = jnp.concatenate([w0 - w2, w1, 2.0 * w2], axis=1)

    p0, p1, p2 = _project(x, wc)

    pad = _EP - _E
    src = jnp.concatenate([edge_index[1], jnp.zeros((pad,), jnp.int32)])
    dst = jnp.concatenate([edge_index[0], jnp.zeros((pad,), jnp.int32)])
    ew = jnp.concatenate([edge_weight, jnp.zeros((pad,), jnp.float32)])
    srcg = (src[None, :]
            + (jnp.arange(_G, dtype=jnp.int32) * _N)[:, None]
            ).reshape(_G, _NS, _NCH, _CH)
    dstg = dst.reshape(_NS, _NCH, _CH)
    wg = ew.reshape(_NS, _NCH, _CH)

    t = _spmm(p2.reshape(_G * _N, _GW), p1.reshape(_G * _N, _GW),
              srcg, dstg, wg)
    u = _spmm(t, p0.reshape(_G * _N, _GW), srcg, dstg, wg)

    return _finish(u.reshape(_G, _N, 2, _OUT), b.reshape(1, _OUT))
